# Initial kernel scaffold; baseline (speedup 1.0000x reference)
import jax
import jax.numpy as jnp
from jax.experimental import pallas as pl


def kernel(x, edge_index, batch, W_conv, b_conv, W_lin, b_lin, W_out, b_out):
    raise NotImplementedError("write your pallas kernel here")


# speedup vs baseline: 8.7254x; 8.7254x over previous
import functools

import jax
import jax.numpy as jnp
from jax import lax
from jax.experimental import pallas as pl
from jax.experimental.pallas import tpu as pltpu
from jax.experimental.pallas import tpu_sc as plsc

N_TILES = 32
CHUNK = 128
LANES = 16
N_GRAPHS = 64


def _mesh():
    return plsc.VectorSubcoreMesh(core_axis_name="c", subcore_axis_name="s")


def _deg_body(dst_hbm, z_hbm, ones_hbm, out_hbm,
              dst_v, msg_v, acc_sh, *, nchunk, n_sub):
    c = lax.axis_index("c")
    s = lax.axis_index("s")
    t = s * 2 + c

    pltpu.sync_copy(dst_hbm.at[t], dst_v)

    pltpu.sync_copy(z_hbm, msg_v)
    for m in range(n_sub):
        pltpu.sync_copy(msg_v, acc_sh.at[pl.ds((s * n_sub + m) * CHUNK, CHUNK)])
    pltpu.sync_copy(ones_hbm, msg_v)
    plsc.subcore_barrier()

    def body(j, carry):
        pltpu.sync_copy(msg_v, acc_sh.at[dst_v.at[j]], add=True)
        return carry

    lax.fori_loop(0, nchunk, body, 0)
    plsc.subcore_barrier()

    for m in range(n_sub):
        pltpu.sync_copy(acc_sh.at[pl.ds((s * n_sub + m) * CHUNK, CHUNK)], msg_v)
        pltpu.sync_copy(msg_v, out_hbm.at[c, s * n_sub + m])


def _make_deg_kernel(nchunk, n_pad, d):
    n_sub = n_pad // (16 * CHUNK)
    body = functools.partial(_deg_body, nchunk=nchunk, n_sub=n_sub)
    return pl.kernel(
        body,
        mesh=_mesh(),
        out_type=jax.ShapeDtypeStruct((2, n_pad // CHUNK, CHUNK, d), jnp.float32),
        scratch_types=[
            pltpu.VMEM((nchunk, CHUNK), jnp.int32),
            pltpu.VMEM((CHUNK, d), jnp.float32),
            pltpu.VMEM_SHARED((n_pad, d), jnp.float32),
        ],
    )


def _msg_body(y_hbm, src_hbm, dst_hbm, z_hbm, out_hbm,
              src_v, dst_v, msg_v, acc_sh, sem, *, nchunk, n_sub):
    c = lax.axis_index("c")
    s = lax.axis_index("s")
    t = s * 2 + c

    pltpu.sync_copy(src_hbm.at[t], src_v)
    pltpu.sync_copy(dst_hbm.at[t], dst_v)

    pltpu.sync_copy(z_hbm, msg_v)
    for m in range(n_sub):
        pltpu.sync_copy(msg_v, acc_sh.at[pl.ds((s * n_sub + m) * CHUNK, CHUNK)])
    plsc.subcore_barrier()

    def body(j, carry):
        pltpu.async_copy(y_hbm.at[src_v.at[j]], msg_v, sem).wait()
        pltpu.sync_copy(msg_v, acc_sh.at[dst_v.at[j]], add=True)
        return carry

    lax.fori_loop(0, nchunk, body, 0)
    plsc.subcore_barrier()

    for m in range(n_sub):
        pltpu.sync_copy(acc_sh.at[pl.ds((s * n_sub + m) * CHUNK, CHUNK)], msg_v)
        pltpu.sync_copy(msg_v, out_hbm.at[c, s * n_sub + m])


def _make_msg_kernel(nchunk, n_pad, d):
    n_sub = n_pad // (16 * CHUNK)
    body = functools.partial(_msg_body, nchunk=nchunk, n_sub=n_sub)
    return pl.kernel(
        body,
        mesh=_mesh(),
        out_type=jax.ShapeDtypeStruct((2, n_pad // CHUNK, CHUNK, d), jnp.float32),
        scratch_types=[
            pltpu.VMEM((nchunk, CHUNK), jnp.int32),
            pltpu.VMEM((nchunk, CHUNK), jnp.int32),
            pltpu.VMEM((CHUNK, d), jnp.float32),
            pltpu.VMEM_SHARED((n_pad, d), jnp.float32),
            pltpu.SemaphoreType.DMA,
        ],
    )


def _tc_deg_body(acc_ref, dinv_ref, *, d):
    deg = jnp.sum(acc_ref[0] + acc_ref[1], axis=-1, keepdims=True) * (1.0 / d)
    dinv_ref[...] = lax.rsqrt(deg + 1.0)


def _tc_first_body(x_ref, w_ref, dinv_ref, y_ref):
    y_ref[...] = jnp.dot(x_ref[...], w_ref[...],
                         preferred_element_type=jnp.float32, precision=lax.Precision.HIGHEST) * dinv_ref[...]


def _tc_mid_body(acc_ref, y_ref, dinv_ref, w_ref, b_ref, yo_ref, *, n):
    d = dinv_ref[...]
    agg = acc_ref[0, :n, :] + acc_ref[1, :n, :] + y_ref[...]
    h = jax.nn.sigmoid(agg * d + b_ref[...])
    yo_ref[...] = jnp.dot(h, w_ref[...], preferred_element_type=jnp.float32, precision=lax.Precision.HIGHEST) * d


def _tc_final_body(acc_ref, y_ref, dinv_ref, b_ref, batch_ref,
                   wl_ref, bl_ref, wo_ref, bo_ref, out_ref, *, n):
    d = dinv_ref[...]
    agg = acc_ref[0, :n, :] + acc_ref[1, :n, :] + y_ref[...]
    h = jax.nn.sigmoid(agg * d + b_ref[...])
    gid = lax.broadcasted_iota(jnp.int32, (N_GRAPHS, n), 0)
    onehot = (gid == batch_ref[...]).astype(jnp.float32)
    sums = jnp.dot(onehot, h, preferred_element_type=jnp.float32, precision=lax.Precision.HIGHEST)
    counts = jnp.sum(onehot, axis=1, keepdims=True)
    pooled = sums / jnp.maximum(counts, 1.0)
    for i in range(wl_ref.shape[0]):
        pooled = jnp.maximum(
            jnp.dot(pooled, wl_ref[i], preferred_element_type=jnp.float32, precision=lax.Precision.HIGHEST)
            + bl_ref[pl.ds(i, 1), :], 0.0)
    out_ref[...] = (jnp.dot(pooled, wo_ref[...],
                            preferred_element_type=jnp.float32, precision=lax.Precision.HIGHEST) + bo_ref[...])


def kernel(x, edge_index, batch, W_conv, b_conv, W_lin, b_lin, W_out, b_out):
    n, d = x.shape
    e = edge_index.shape[1]
    n_convs = W_conv.shape[0]

    nchunk = -(-e // (N_TILES * CHUNK))
    e_pad = N_TILES * CHUNK * nchunk
    n_pad = -(-(n + 1) // (16 * CHUNK)) * (16 * CHUNK)

    src = edge_index[0].astype(jnp.int32)
    dst = edge_index[1].astype(jnp.int32)
    src_p = jnp.concatenate(
        [src, jnp.zeros((e_pad - e,), jnp.int32)]).reshape(N_TILES, nchunk, CHUNK)
    dst_p = jnp.concatenate(
        [dst, jnp.full((e_pad - e,), n, jnp.int32)]).reshape(N_TILES, nchunk, CHUNK)
    ones_msg = jnp.ones((CHUNK, d), jnp.float32)
    z_msg = jnp.zeros((CHUNK, d), jnp.float32)
    batch2 = batch.astype(jnp.int32).reshape(1, n)

    deg_kernel = _make_deg_kernel(nchunk, n_pad, d)
    msg_kernel = _make_msg_kernel(nchunk, n_pad, d)

    accd = deg_kernel(dst_p, z_msg, ones_msg).reshape(2, n_pad, d)
    dinv_pad = pl.pallas_call(
        functools.partial(_tc_deg_body, d=d),
        out_shape=jax.ShapeDtypeStruct((n_pad, 1), jnp.float32),
    )(accd)
    dinv_col = dinv_pad[:n]

    y = pl.pallas_call(
        _tc_first_body,
        out_shape=jax.ShapeDtypeStruct((n, d), jnp.float32),
    )(x, W_conv[0], dinv_col)

    tc_mid = pl.pallas_call(
        functools.partial(_tc_mid_body, n=n),
        out_shape=jax.ShapeDtypeStruct((n, d), jnp.float32),
    )
    for i in range(1, n_convs):
        acc = msg_kernel(y, src_p, dst_p, z_msg).reshape(2, n_pad, d)
        y = tc_mid(acc, y, dinv_col, W_conv[i], b_conv[i - 1][None, :])

    acc = msg_kernel(y, src_p, dst_p, z_msg).reshape(2, n_pad, d)

    out = pl.pallas_call(
        functools.partial(_tc_final_body, n=n),
        out_shape=jax.ShapeDtypeStruct((N_GRAPHS, 1), jnp.float32),
    )(acc, y, dinv_col, b_conv[n_convs - 1][None, :], batch2,
      W_lin, b_lin, W_out, b_out.reshape(1, 1))
    return out
